# baseline (speedup 1.0000x reference)
import functools

import jax
import jax.numpy as jnp
from jax import lax
from jax.experimental import pallas as pl
from jax.experimental.pallas import tpu as pltpu
from jax.experimental.pallas import tpu_sc as plsc

N_EMBED = 1024
MAX_SEQ_LEN = 8192
BATCH = 4


def _make_sc_broadcast():
    info = plsc.get_sparse_core_info()
    num_cores, num_subcores = info.num_cores, info.num_subcores
    num_workers = num_cores * num_subcores
    rows_per_worker = MAX_SEQ_LEN // num_workers

    mesh = plsc.VectorSubcoreMesh(core_axis_name="c", subcore_axis_name="s")

    @functools.partial(
        pl.kernel,
        mesh=mesh,
        out_type=jax.ShapeDtypeStruct((BATCH, MAX_SEQ_LEN, N_EMBED), jnp.float32),
    )
    def broadcast_rows(table_hbm, out_hbm):
        wid = lax.axis_index("s") * num_cores + lax.axis_index("c")
        base = wid * rows_per_worker
        for b in range(BATCH):
            pltpu.sync_copy(
                table_hbm.at[pl.ds(base, rows_per_worker)],
                out_hbm.at[b, pl.ds(base, rows_per_worker)],
            )

    return broadcast_rows


_sc_broadcast = _make_sc_broadcast()


def kernel(input_ids, pos_table):
    del input_ids
    return _sc_broadcast(pos_table)


# speedup vs baseline: 55.7100x; 55.7100x over previous
import functools

import jax
import jax.numpy as jnp
from jax import lax
from jax.experimental import pallas as pl
from jax.experimental.pallas import tpu as pltpu
from jax.experimental.pallas import tpu_sc as plsc

N_EMBED = 1024
MAX_SEQ_LEN = 8192
BATCH = 4


def _make_sc_broadcast():
    info = plsc.get_sparse_core_info()
    num_cores, num_subcores = info.num_cores, info.num_subcores
    num_workers = num_cores * num_subcores
    rows_per_worker = MAX_SEQ_LEN // num_workers

    mesh = plsc.VectorSubcoreMesh(core_axis_name="c", subcore_axis_name="s")

    chunk_rows = 64
    num_chunks = rows_per_worker // chunk_rows

    @functools.partial(
        pl.kernel,
        mesh=mesh,
        out_type=jax.ShapeDtypeStruct((BATCH, MAX_SEQ_LEN, N_EMBED), jnp.float32),
        scratch_types=[pltpu.VMEM((chunk_rows, N_EMBED), jnp.float32)],
    )
    def broadcast_rows(table_hbm, out_hbm, buf):
        wid = lax.axis_index("s") * num_cores + lax.axis_index("c")
        base = wid * rows_per_worker

        def body(i, carry):
            row0 = base + i * chunk_rows
            pltpu.sync_copy(table_hbm.at[pl.ds(row0, chunk_rows)], buf)
            for b in range(BATCH):
                pltpu.sync_copy(buf, out_hbm.at[b, pl.ds(row0, chunk_rows)])
            return carry

        lax.fori_loop(0, num_chunks, body, 0)

    return broadcast_rows


_sc_broadcast = _make_sc_broadcast()


def kernel(input_ids, pos_table):
    del input_ids
    return _sc_broadcast(pos_table)
